# baseline (speedup 1.0000x reference)
import functools

import jax
import jax.numpy as jnp
from jax import lax
from jax.experimental import pallas as pl
from jax.experimental.pallas import tpu as pltpu
from jax.experimental.pallas import tpu_sc as plsc

D_MODEL = 64
SCALE = 8.0
NC, NS = 2, 16
NW = NC * NS
R = 128


@functools.lru_cache(maxsize=None)
def _make_kernel(B):
    b_per_w = B // NW
    n_chunks = b_per_w // R
    mesh = plsc.VectorSubcoreMesh(
        core_axis_name="c", subcore_axis_name="s", num_cores=NC, num_subcores=NS
    )

    @functools.partial(
        pl.kernel,
        out_type=jax.ShapeDtypeStruct((B, D_MODEL), jnp.float32),
        mesh=mesh,
        scratch_types=[
            pltpu.VMEM((n_chunks, R), jnp.int32),
            pltpu.VMEM((R, D_MODEL), jnp.float32),
            pltpu.SemaphoreType.DMA,
        ],
        compiler_params=pltpu.CompilerParams(use_tc_tiling_on_sc=False),
    )
    def emb(idx_hbm, table_hbm, out_hbm, idx_v, rows_v, sem):
        wid = lax.axis_index("s") * NC + lax.axis_index("c")
        pltpu.sync_copy(idx_hbm.at[wid], idx_v)

        def chunk(j, carry):
            pltpu.async_copy(table_hbm.at[idx_v.at[j]], rows_v, sem).wait()

            def row(r, c2):
                for kk in range(D_MODEL // 16):
                    sl = pl.ds(kk * 16, 16)
                    rows_v[r, sl] = rows_v[r, sl] * SCALE
                return c2

            lax.fori_loop(0, R, row, 0)
            pltpu.sync_copy(rows_v, out_hbm.at[pl.ds(wid * b_per_w + j * R, R)])
            return carry

        lax.fori_loop(0, n_chunks, chunk, 0)

    return emb


def kernel(x, table):
    B = x.shape[0] * x.shape[1]
    idx = x.reshape(NW, B // NW // R, R).astype(jnp.int32)
    out = _make_kernel(B)(idx, table)
    return out.reshape(x.shape[0], x.shape[1], D_MODEL)


# speedup vs baseline: 1.2101x; 1.2101x over previous
import functools

import jax
import jax.numpy as jnp
from jax import lax
from jax.experimental import pallas as pl
from jax.experimental.pallas import tpu as pltpu
from jax.experimental.pallas import tpu_sc as plsc

D_MODEL = 64
SCALE = 8.0
NC, NS = 2, 16
NW = NC * NS
R = 128
NBUF = 4


@functools.lru_cache(maxsize=None)
def _make_kernel(B):
    b_per_w = B // NW
    n_chunks = b_per_w // R
    n_super = n_chunks // NBUF
    mesh = plsc.VectorSubcoreMesh(
        core_axis_name="c", subcore_axis_name="s", num_cores=NC, num_subcores=NS
    )

    @functools.partial(
        pl.kernel,
        out_type=jax.ShapeDtypeStruct((B, D_MODEL), jnp.float32),
        mesh=mesh,
        scratch_types=[
            pltpu.VMEM((n_chunks, R), jnp.int32),
            [pltpu.VMEM((R, D_MODEL), jnp.float32)] * NBUF,
            [pltpu.VMEM((R, D_MODEL), jnp.float32)] * NBUF,
            [pltpu.SemaphoreType.DMA] * NBUF,
            [pltpu.SemaphoreType.DMA] * NBUF,
        ],
        compiler_params=pltpu.CompilerParams(use_tc_tiling_on_sc=False),
    )
    def emb(idx_hbm, table_hbm, out_hbm, idx_v, gbuf, wbuf, gsem, wsem):
        wid = lax.axis_index("s") * NC + lax.axis_index("c")
        base = wid * b_per_w
        pltpu.sync_copy(idx_hbm.at[wid], idx_v)

        for b in range(NBUF):
            pltpu.async_copy(table_hbm.at[idx_v.at[b]], gbuf[b], gsem[b])

        def superstep(t, carry):
            for b in range(NBUF):
                k = t * NBUF + b
                pltpu.make_async_copy(
                    table_hbm.at[idx_v.at[k]], gbuf[b], gsem[b]
                ).wait()

                @pl.when(t != 0)
                def _():
                    pltpu.make_async_copy(
                        wbuf[b], out_hbm.at[pl.ds(base, R)], wsem[b]
                    ).wait()

                gb, wb = gbuf[b], wbuf[b]

                @plsc.parallel_loop(0, R, step=1, unroll=8)
                def _(r):
                    for kk in range(D_MODEL // 16):
                        sl = pl.ds(kk * 16, 16)
                        wb[r, sl] = gb[r, sl] * SCALE

                pltpu.async_copy(
                    wbuf[b], out_hbm.at[pl.ds(base + k * R, R)], wsem[b]
                )

                @pl.when(t != n_super - 1)
                def _():
                    pltpu.async_copy(
                        table_hbm.at[idx_v.at[k + NBUF]], gbuf[b], gsem[b]
                    )

            return carry

        lax.fori_loop(0, n_super, superstep, 0)

        for b in range(NBUF):
            pltpu.make_async_copy(
                wbuf[b], out_hbm.at[pl.ds(base, R)], wsem[b]
            ).wait()

    return emb


def kernel(x, table):
    B = x.shape[0] * x.shape[1]
    idx = x.reshape(NW, B // NW // R, R).astype(jnp.int32)
    out = _make_kernel(B)(idx, table)
    return out.reshape(x.shape[0], x.shape[1], D_MODEL)
